# baseline (speedup 1.0000x reference)
import functools

import jax
import jax.numpy as jnp
from jax import lax
from jax.experimental import pallas as pl
from jax.experimental.pallas import tpu as pltpu
from jax.experimental.pallas import tpu_sc as plsc

_D = 256
_V = 8192
_TB = 256


def _argmin_body(x_ref, w_ref, out_ref, wn_ref, wb_ref, ids_ref):
    @pl.when(pl.program_id(0) == 0)
    def _():
        wb_ref[...] = w_ref[...].astype(jnp.bfloat16).T
        wt = wb_ref[...].astype(jnp.float32)
        wn_ref[...] = jnp.sum(wt * wt, axis=0, keepdims=True)
        ids_ref[...] = lax.broadcasted_iota(jnp.int32, (_TB, _V), 1)

    xb = (x_ref[...] * -2.0).astype(jnp.bfloat16)
    dots = lax.dot_general(xb, wb_ref[...], (((1,), (0,)), ((), ())),
                           preferred_element_type=jnp.float32)
    scores = dots + wn_ref[...]
    ids = ids_ref[...]
    key = (lax.bitcast_convert_type(scores, jnp.int32) & jnp.int32(-_V)) | ids
    keyf = lax.bitcast_convert_type(key, jnp.float32)
    mf = jnp.min(keyf, axis=1)
    idx = lax.bitcast_convert_type(mf, jnp.int32) & jnp.int32(_V - 1)
    out_ref[...] = idx.reshape(1, 1, _TB)


def _tc_argmin(flat, weight):
    n = flat.shape[0]
    grid = n // _TB
    out = pl.pallas_call(
        _argmin_body,
        grid=(grid,),
        in_specs=[
            pl.BlockSpec((_TB, _D), lambda i: (i, 0)),
            pl.BlockSpec((_V, _D), lambda i: (0, 0)),
        ],
        out_specs=pl.BlockSpec((1, 1, _TB), lambda i: (i, 0, 0)),
        out_shape=jax.ShapeDtypeStruct((grid, 1, _TB), jnp.int32),
        scratch_shapes=[pltpu.VMEM((1, _V), jnp.float32),
                        pltpu.VMEM((_D, _V), jnp.bfloat16),
                        pltpu.VMEM((_TB, _V), jnp.int32)],
    )(flat, weight)
    return out.reshape(n)


_NW = 32
_CHUNK = 96
_NCHUNK = 3


def _sc_body(w_hbm, x_hbm, idx_hbm, out_hbm, idx0, idx1, idx2, rows_v, x_v,
             acc_v, gsem0, gsem1, xsem0, xsem1):
    wid = lax.axis_index("s") * 2 + lax.axis_index("c")
    base0 = wid * _NCHUNK * _CHUNK
    idxs = (idx0, idx1, idx2)
    for c in range(_NCHUNK):
        pltpu.sync_copy(idx_hbm.at[pl.ds(base0 + c * _CHUNK, _CHUNK)], idxs[c])
    gsems = (gsem0, gsem1)
    xsems = (xsem0, xsem1)

    def start(c):
        b = c & 1
        g = pltpu.async_copy(w_hbm.at[idxs[c]], rows_v.at[b], gsems[b])
        x = pltpu.async_copy(x_hbm.at[pl.ds(base0 + c * _CHUNK, _CHUNK)],
                             x_v.at[b], xsems[b])
        return g, x

    acc = jnp.zeros((16,), jnp.float32)
    pending = start(0)
    for c in range(_NCHUNK):
        nxt = start(c + 1) if c + 1 < _NCHUNK else None
        pending[0].wait()
        pending[1].wait()
        b = c & 1

        def body(t, a, _b=b):
            for j in range(_D // 16):
                d = (x_v[_b, t, pl.ds(j * 16, 16)]
                     - rows_v[_b, t, pl.ds(j * 16, 16)])
                a = a + d * d
            return a

        acc = lax.fori_loop(0, _CHUNK, body, acc)
        pending = nxt
    acc_v[...] = acc
    pltpu.sync_copy(acc_v, out_hbm.at[wid])


def _sc_residual(weight, flat, idx):
    mesh = plsc.VectorSubcoreMesh(core_axis_name="c", subcore_axis_name="s")
    k = functools.partial(
        pl.kernel,
        mesh=mesh,
        out_type=jax.ShapeDtypeStruct((_NW, 16), jnp.float32),
        scratch_types=[
            pltpu.VMEM((_CHUNK,), jnp.int32),
            pltpu.VMEM((_CHUNK,), jnp.int32),
            pltpu.VMEM((_CHUNK,), jnp.int32),
            pltpu.VMEM((2, _CHUNK, _D), jnp.float32),
            pltpu.VMEM((2, _CHUNK, _D), jnp.float32),
            pltpu.VMEM((16,), jnp.float32),
            pltpu.SemaphoreType.DMA,
            pltpu.SemaphoreType.DMA,
            pltpu.SemaphoreType.DMA,
            pltpu.SemaphoreType.DMA,
        ],
    )(_sc_body)
    return k(weight, flat, idx)


def kernel(inputs, weight):
    flat = inputs.reshape(-1, _D)
    n = flat.shape[0]
    idx = _tc_argmin(flat, weight)
    partials = _sc_residual(weight, flat, idx)
    return jnp.sum(partials) / jnp.float32(n * _D)


# speedup vs baseline: 1.0695x; 1.0695x over previous
import functools

import jax
import jax.numpy as jnp
from jax import lax
from jax.experimental import pallas as pl
from jax.experimental.pallas import tpu as pltpu
from jax.experimental.pallas import tpu_sc as plsc

_D = 256
_V = 8192
_TB = 512


def _argmin_body(x_ref, w_ref, out_ref, wn_ref, wb_ref):
    @pl.when(pl.program_id(0) == 0)
    def _():
        wb_ref[...] = w_ref[...].astype(jnp.bfloat16).T
        wt = wb_ref[...].astype(jnp.float32)
        wn_ref[...] = jnp.sum(wt * wt, axis=0, keepdims=True)

    xb = (x_ref[...] * -2.0).astype(jnp.bfloat16)
    dots = lax.dot_general(xb, wb_ref[...], (((1,), (0,)), ((), ())),
                           preferred_element_type=jnp.float32)
    scores = dots + wn_ref[...]
    ids = lax.broadcasted_iota(jnp.int32, scores.shape, 1)
    key = (lax.bitcast_convert_type(scores, jnp.int32) & jnp.int32(-_V)) | ids
    keyf = lax.bitcast_convert_type(key, jnp.float32)
    mf = jnp.min(keyf, axis=1)
    idx = lax.bitcast_convert_type(mf, jnp.int32) & jnp.int32(_V - 1)
    out_ref[...] = idx.reshape(1, 1, _TB)


def _tc_argmin(flat, weight):
    n = flat.shape[0]
    grid = n // _TB
    out = pl.pallas_call(
        _argmin_body,
        grid=(grid,),
        in_specs=[
            pl.BlockSpec((_TB, _D), lambda i: (i, 0)),
            pl.BlockSpec((_V, _D), lambda i: (0, 0)),
        ],
        out_specs=pl.BlockSpec((1, 1, _TB), lambda i: (i, 0, 0)),
        out_shape=jax.ShapeDtypeStruct((grid, 1, _TB), jnp.int32),
        scratch_shapes=[pltpu.VMEM((1, _V), jnp.float32),
                        pltpu.VMEM((_D, _V), jnp.bfloat16)],
    )(flat, weight)
    return out.reshape(n)


_NW = 32
_CHUNK = 96
_NCHUNK = 3


def _sc_body(w_hbm, x_hbm, idx_hbm, out_hbm, idx0, idx1, idx2, rows_v, x_v,
             acc_v, gsem0, gsem1, xsem0, xsem1):
    wid = lax.axis_index("s") * 2 + lax.axis_index("c")
    base0 = wid * _NCHUNK * _CHUNK
    idxs = (idx0, idx1, idx2)
    for c in range(_NCHUNK):
        pltpu.sync_copy(idx_hbm.at[pl.ds(base0 + c * _CHUNK, _CHUNK)], idxs[c])
    gsems = (gsem0, gsem1)
    xsems = (xsem0, xsem1)

    def start(c):
        b = c & 1
        g = pltpu.async_copy(w_hbm.at[idxs[c]], rows_v.at[b], gsems[b])
        x = pltpu.async_copy(x_hbm.at[pl.ds(base0 + c * _CHUNK, _CHUNK)],
                             x_v.at[b], xsems[b])
        return g, x

    acc = jnp.zeros((16,), jnp.float32)
    pending = start(0)
    for c in range(_NCHUNK):
        nxt = start(c + 1) if c + 1 < _NCHUNK else None
        pending[0].wait()
        pending[1].wait()
        b = c & 1

        def body(t, a, _b=b):
            for j in range(_D // 16):
                d = (x_v[_b, t, pl.ds(j * 16, 16)]
                     - rows_v[_b, t, pl.ds(j * 16, 16)])
                a = a + d * d
            return a

        acc = lax.fori_loop(0, _CHUNK, body, acc)
        pending = nxt
    acc_v[...] = acc
    pltpu.sync_copy(acc_v, out_hbm.at[wid])


def _sc_residual(weight, flat, idx):
    mesh = plsc.VectorSubcoreMesh(core_axis_name="c", subcore_axis_name="s")
    k = functools.partial(
        pl.kernel,
        mesh=mesh,
        out_type=jax.ShapeDtypeStruct((_NW, 16), jnp.float32),
        scratch_types=[
            pltpu.VMEM((_CHUNK,), jnp.int32),
            pltpu.VMEM((_CHUNK,), jnp.int32),
            pltpu.VMEM((_CHUNK,), jnp.int32),
            pltpu.VMEM((2, _CHUNK, _D), jnp.float32),
            pltpu.VMEM((2, _CHUNK, _D), jnp.float32),
            pltpu.VMEM((16,), jnp.float32),
            pltpu.SemaphoreType.DMA,
            pltpu.SemaphoreType.DMA,
            pltpu.SemaphoreType.DMA,
            pltpu.SemaphoreType.DMA,
        ],
    )(_sc_body)
    return k(weight, flat, idx)


def kernel(inputs, weight):
    flat = inputs.reshape(-1, _D)
    n = flat.shape[0]
    idx = _tc_argmin(flat, weight)
    partials = _sc_residual(weight, flat, idx)
    return jnp.sum(partials) / jnp.float32(n * _D)
